# Initial kernel scaffold; baseline (speedup 1.0000x reference)
import jax
import jax.numpy as jnp
from jax.experimental import pallas as pl


def kernel(keys, values, key_centroids, value_centroids):
    raise NotImplementedError("write your pallas kernel here")


# speedup vs baseline: 3.1756x; 3.1756x over previous
import jax
import jax.numpy as jnp
from jax.experimental import pallas as pl
from jax.experimental.pallas import tpu as pltpu

_NUM_CLUSTERS = 1024
_HEAD_DIM = 128
_LR = 0.01
_TBLK = 512


def _km_step(kf_ref, vf_ref, kc_ref, vc_ref,
             idx_ref, kc_out_ref, vc_out_ref,
             acc_ref, cnt_ref):
    step = pl.program_id(0)
    nsteps = pl.num_programs(0)

    @pl.when(step == 0)
    def _init():
        acc_ref[...] = jnp.zeros_like(acc_ref)
        cnt_ref[...] = jnp.zeros_like(cnt_ref)

    kblk = kf_ref[...]
    vblk = vf_ref[...]
    kc = kc_ref[...]

    c2 = jnp.sum(kc * kc, axis=1, keepdims=True)
    xcT = jax.lax.dot_general(kc, kblk, (((1,), (1,)), ((), ())),
                              preferred_element_type=jnp.float32)
    d2T = c2 - 2.0 * xcT

    mT = jnp.min(d2T, axis=0, keepdims=True)
    row = jax.lax.broadcasted_iota(jnp.int32, d2T.shape, 0)
    idx_row = jnp.min(jnp.where(d2T == mT, row, _NUM_CLUSTERS),
                      axis=0, keepdims=True)
    idx_ref[...] = idx_row[None, :, :]

    ohT = (row == idx_row).astype(jnp.float32)
    kv = jnp.concatenate([kblk, vblk], axis=1)
    acc_ref[...] += jax.lax.dot_general(ohT, kv, (((1,), (0,)), ((), ())),
                                        preferred_element_type=jnp.float32)
    cnt_ref[...] += jnp.sum(ohT, axis=1, keepdims=True)

    @pl.when(step == nsteps - 1)
    def _fin():
        cnt = cnt_ref[...]
        denom = jnp.maximum(cnt, 1.0)
        acc = acc_ref[...]
        kmean = acc[:, :_HEAD_DIM] / denom
        vmean = acc[:, _HEAD_DIM:] / denom
        ne = cnt > 0.0
        kc0 = kc_ref[...]
        vc0 = vc_ref[...]
        kc_out_ref[...] = jnp.where(ne, (1.0 - _LR) * kc0 + _LR * kmean, kc0)
        vc_out_ref[...] = jnp.where(ne, (1.0 - _LR) * vc0 + _LR * vmean, vc0)


def _build_call(nb, interpret=False):
    return pl.pallas_call(
        _km_step,
        grid=(nb,),
        in_specs=[
            pl.BlockSpec((_TBLK, _HEAD_DIM), lambda i: (i, 0)),
            pl.BlockSpec((_TBLK, _HEAD_DIM), lambda i: (i, 0)),
            pl.BlockSpec((_NUM_CLUSTERS, _HEAD_DIM), lambda i: (0, 0)),
            pl.BlockSpec((_NUM_CLUSTERS, _HEAD_DIM), lambda i: (0, 0)),
        ],
        out_specs=[
            pl.BlockSpec((1, 1, _TBLK), lambda i: (i, 0, 0)),
            pl.BlockSpec((_NUM_CLUSTERS, _HEAD_DIM), lambda i: (0, 0)),
            pl.BlockSpec((_NUM_CLUSTERS, _HEAD_DIM), lambda i: (0, 0)),
        ],
        out_shape=[
            jax.ShapeDtypeStruct((nb, 1, _TBLK), jnp.int32),
            jax.ShapeDtypeStruct((_NUM_CLUSTERS, _HEAD_DIM), jnp.float32),
            jax.ShapeDtypeStruct((_NUM_CLUSTERS, _HEAD_DIM), jnp.float32),
        ],
        scratch_shapes=[
            pltpu.VMEM((_NUM_CLUSTERS, 2 * _HEAD_DIM), jnp.float32),
            pltpu.VMEM((_NUM_CLUSTERS, 1), jnp.float32),
        ],
        interpret=interpret,
    )


def kernel(keys, values, key_centroids, value_centroids):
    n = keys.shape[0] * keys.shape[1] * keys.shape[2]
    kf = keys.reshape(n, _HEAD_DIM)
    vf = values.reshape(n, _HEAD_DIM)
    nb = n // _TBLK
    idx3, kc_new, vc_new = _build_call(nb)(kf, vf, key_centroids,
                                           value_centroids)
    cluster_idx = idx3.reshape(keys.shape[:-1])
    return (cluster_idx, kc_new, vc_new)


# speedup vs baseline: 6.4978x; 2.0462x over previous
import jax
import jax.numpy as jnp
from jax.experimental import pallas as pl
from jax.experimental.pallas import tpu as pltpu

_NUM_CLUSTERS = 1024
_HEAD_DIM = 128
_LR = 0.01
_TBLK = 4096


def _km_step(kf_ref, vf_ref, kc_ref, vc_ref,
             idx_ref, kc_out_ref, vc_out_ref,
             acc_ref, kc2_ref, c2_ref, cnt_ref):
    step = pl.program_id(0)
    nsteps = pl.num_programs(0)

    @pl.when(step == 0)
    def _init():
        acc_ref[...] = jnp.zeros_like(acc_ref)
        cnt_ref[...] = jnp.zeros_like(cnt_ref)
        kc = kc_ref[...]
        kc2_ref[...] = kc * -2.0
        c2_ref[...] = jnp.sum(kc * kc, axis=1, keepdims=True)

    kblk = kf_ref[...]
    vblk = vf_ref[...]

    xcT = jax.lax.dot_general(kc2_ref[...], kblk, (((1,), (1,)), ((), ())),
                              preferred_element_type=jnp.float32)
    d2T = c2_ref[...] + xcT

    mT = jnp.min(d2T, axis=0, keepdims=True)
    eq = d2T == mT
    ohT = eq.astype(jnp.float32)

    row = jax.lax.broadcasted_iota(jnp.int32, d2T.shape, 0)
    idx_row = jnp.min(jnp.where(eq, row, _NUM_CLUSTERS),
                      axis=0, keepdims=True)
    idx_ref[...] = idx_row[None, :, :]

    kv = jnp.concatenate([kblk, vblk], axis=1)
    acc_ref[...] += jax.lax.dot_general(ohT, kv, (((1,), (0,)), ((), ())),
                                        preferred_element_type=jnp.float32)
    cnt_ref[...] += jnp.sum(ohT, axis=1, keepdims=True)

    @pl.when(step == nsteps - 1)
    def _fin():
        acc = acc_ref[...]
        cnt = cnt_ref[...]
        denom = jnp.maximum(cnt, 1.0)
        kmean = acc[:, :_HEAD_DIM] / denom
        vmean = acc[:, _HEAD_DIM:] / denom
        ne = cnt > 0.0
        kc0 = kc_ref[...]
        vc0 = vc_ref[...]
        kc_out_ref[...] = jnp.where(ne, (1.0 - _LR) * kc0 + _LR * kmean, kc0)
        vc_out_ref[...] = jnp.where(ne, (1.0 - _LR) * vc0 + _LR * vmean, vc0)


def _build_call(nb, interpret=False):
    return pl.pallas_call(
        _km_step,
        grid=(nb,),
        in_specs=[
            pl.BlockSpec((_TBLK, _HEAD_DIM), lambda i: (i, 0)),
            pl.BlockSpec((_TBLK, _HEAD_DIM), lambda i: (i, 0)),
            pl.BlockSpec((_NUM_CLUSTERS, _HEAD_DIM), lambda i: (0, 0)),
            pl.BlockSpec((_NUM_CLUSTERS, _HEAD_DIM), lambda i: (0, 0)),
        ],
        out_specs=[
            pl.BlockSpec((1, 1, _TBLK), lambda i: (i, 0, 0)),
            pl.BlockSpec((_NUM_CLUSTERS, _HEAD_DIM), lambda i: (0, 0)),
            pl.BlockSpec((_NUM_CLUSTERS, _HEAD_DIM), lambda i: (0, 0)),
        ],
        out_shape=[
            jax.ShapeDtypeStruct((nb, 1, _TBLK), jnp.int32),
            jax.ShapeDtypeStruct((_NUM_CLUSTERS, _HEAD_DIM), jnp.float32),
            jax.ShapeDtypeStruct((_NUM_CLUSTERS, _HEAD_DIM), jnp.float32),
        ],
        scratch_shapes=[
            pltpu.VMEM((_NUM_CLUSTERS, 2 * _HEAD_DIM), jnp.float32),
            pltpu.VMEM((_NUM_CLUSTERS, _HEAD_DIM), jnp.float32),
            pltpu.VMEM((_NUM_CLUSTERS, 1), jnp.float32),
            pltpu.VMEM((_NUM_CLUSTERS, 1), jnp.float32),
        ],
        interpret=interpret,
    )


def kernel(keys, values, key_centroids, value_centroids):
    n = keys.shape[0] * keys.shape[1] * keys.shape[2]
    kf = keys.reshape(n, _HEAD_DIM)
    vf = values.reshape(n, _HEAD_DIM)
    nb = n // _TBLK
    idx3, kc_new, vc_new = _build_call(nb)(kf, vf, key_centroids,
                                           value_centroids)
    cluster_idx = idx3.reshape(keys.shape[:-1])
    return (cluster_idx, kc_new, vc_new)
